# baseline (speedup 1.0000x reference)
import jax
import jax.numpy as jnp
from jax.experimental import pallas as pl

DIM = 1280
GROUPS = 2
NQ = 8
CB_SIZE = 1024
CB_DIM = 256
GDIM = DIM // GROUPS
B = 8
L = 1024
N = B * L
T = 128
NT = N // T


def _grvq_kernel(x_ref, winT_ref, bin_ref, woutT_ref, bout_ref, cb_ref,
                 cbT_ref, q_ref, c_ref, lossp_ref):
    res = [None] * GROUPS
    qout = [None] * GROUPS
    idxs = [[] for _ in range(GROUPS)]
    losses = [[] for _ in range(GROUPS)]
    for g in range(GROUPS):
        xg = x_ref[:, g * GDIM:(g + 1) * GDIM]
        res[g] = (jnp.dot(xg, winT_ref[g], preferred_element_type=jnp.float32)
                  + bin_ref[g])
        qout[g] = jnp.zeros_like(res[g])
    for qi in range(NQ):
        for g in range(GROUPS):
            r = res[g]
            cbTq = cbT_ref[g, qi]
            cb2 = jnp.sum(cbTq * cbTq, axis=0, keepdims=True)
            res2 = jnp.sum(r * r, axis=1, keepdims=True)
            dots = jnp.dot(r, cbTq, preferred_element_type=jnp.float32)
            d = (res2 - 2.0 * dots) + cb2
            m = jnp.min(d, axis=1, keepdims=True)
            iota = jax.lax.broadcasted_iota(jnp.int32, d.shape, 1)
            idx = jnp.min(jnp.where(d == m, iota, CB_SIZE), axis=1,
                          keepdims=True)
            oh = (iota == idx).astype(jnp.float32)
            quant = jnp.dot(oh, cb_ref[g, qi],
                            preferred_element_type=jnp.float32,
                            precision=jax.lax.Precision.HIGHEST)
            diff = quant - r
            losses[g].append(jnp.sum(diff * diff).reshape(1, 1))
            qst = r + diff
            qout[g] = qout[g] + qst
            res[g] = r - qst
            idxs[g].append(idx)
    for g in range(GROUPS):
        q_ref[:, g * GDIM:(g + 1) * GDIM] = (
            jnp.dot(qout[g], woutT_ref[g], preferred_element_type=jnp.float32)
            + bout_ref[g])
        c_ref[g] = jnp.concatenate(idxs[g], axis=1)
        lossp_ref[0, g] = jnp.concatenate(losses[g], axis=1)


def kernel(x, W_in, b_in, W_out, b_out, codebooks):
    xr = x.reshape(N, DIM)
    winT = W_in.transpose(0, 2, 1)
    woutT = W_out.transpose(0, 2, 1)
    cbT = codebooks.transpose(0, 1, 3, 2)
    bin3 = b_in.reshape(GROUPS, 1, CB_DIM)
    bout3 = b_out.reshape(GROUPS, 1, GDIM)

    q, c, lossp = pl.pallas_call(
        _grvq_kernel,
        grid=(NT,),
        in_specs=[
            pl.BlockSpec((T, DIM), lambda t: (t, 0)),
            pl.BlockSpec((GROUPS, GDIM, CB_DIM), lambda t: (0, 0, 0)),
            pl.BlockSpec((GROUPS, 1, CB_DIM), lambda t: (0, 0, 0)),
            pl.BlockSpec((GROUPS, CB_DIM, GDIM), lambda t: (0, 0, 0)),
            pl.BlockSpec((GROUPS, 1, GDIM), lambda t: (0, 0, 0)),
            pl.BlockSpec((GROUPS, NQ, CB_SIZE, CB_DIM), lambda t: (0, 0, 0, 0)),
            pl.BlockSpec((GROUPS, NQ, CB_DIM, CB_SIZE), lambda t: (0, 0, 0, 0)),
        ],
        out_specs=[
            pl.BlockSpec((T, DIM), lambda t: (t, 0)),
            pl.BlockSpec((GROUPS, T, NQ), lambda t: (0, t, 0)),
            pl.BlockSpec((1, GROUPS, 1, NQ), lambda t: (t, 0, 0, 0)),
        ],
        out_shape=[
            jax.ShapeDtypeStruct((N, DIM), jnp.float32),
            jax.ShapeDtypeStruct((GROUPS, N, NQ), jnp.int32),
            jax.ShapeDtypeStruct((NT, GROUPS, 1, NQ), jnp.float32),
        ],
    )(xr, winT, bin3, woutT, bout3, codebooks, cbT)

    qg = q.reshape(B, L, DIM)
    cf = c.reshape(GROUPS, B, L, NQ)
    commit = lossp.sum(axis=0).reshape(GROUPS, NQ) / jnp.float32(N * CB_DIM)
    return (qg, cf, commit)


# speedup vs baseline: 1.2734x; 1.2734x over previous
import jax
import jax.numpy as jnp
from jax.experimental import pallas as pl
from jax.experimental.pallas import tpu as pltpu

DIM = 1280
GROUPS = 2
NQ = 8
CB_SIZE = 1024
CB_DIM = 256
GDIM = DIM // GROUPS
B = 8
L = 1024
N = B * L
T = 256
NT = N // T


def _grvq_kernel(x_ref, winT_ref, bin_ref, woutT_ref, bout_ref, cb_ref,
                 cbT_ref, q_ref, c_ref, lossp_ref, cb2s_ref):
    @pl.when(pl.program_id(0) == 0)
    def _init_cb2():
        for g in range(GROUPS):
            for qi in range(NQ):
                cbTq = cbT_ref[g, qi]
                cb2s_ref[g, qi] = jnp.sum(cbTq * cbTq, axis=0, keepdims=True)

    res = [None] * GROUPS
    qout = [None] * GROUPS
    idxs = [[] for _ in range(GROUPS)]
    losses = [[] for _ in range(GROUPS)]
    for g in range(GROUPS):
        xg = x_ref[:, g * GDIM:(g + 1) * GDIM]
        res[g] = (jnp.dot(xg, winT_ref[g], preferred_element_type=jnp.float32)
                  + bin_ref[g])
        qout[g] = jnp.zeros_like(res[g])
    for qi in range(NQ):
        for g in range(GROUPS):
            r = res[g]
            cb2 = cb2s_ref[g, qi]
            res2 = jnp.sum(r * r, axis=1, keepdims=True)
            dots = jnp.dot(r, cbT_ref[g, qi],
                           preferred_element_type=jnp.float32)
            d = (res2 - 2.0 * dots) + cb2
            m = jnp.min(d, axis=1, keepdims=True)
            iota = jax.lax.broadcasted_iota(jnp.int32, d.shape, 1)
            idx = jnp.min(jnp.where(d == m, iota, CB_SIZE), axis=1,
                          keepdims=True)
            oh = (iota == idx).astype(jnp.float32)
            quant = jnp.dot(oh, cb_ref[g, qi],
                            preferred_element_type=jnp.float32,
                            precision=jax.lax.Precision.HIGHEST)
            diff = quant - r
            losses[g].append(jnp.sum(diff * diff).reshape(1, 1))
            qst = r + diff
            qout[g] = qout[g] + qst
            res[g] = r - qst
            idxs[g].append(idx)
    for g in range(GROUPS):
        q_ref[:, g * GDIM:(g + 1) * GDIM] = (
            jnp.dot(qout[g], woutT_ref[g], preferred_element_type=jnp.float32)
            + bout_ref[g])
        c_ref[g] = jnp.concatenate(idxs[g], axis=1)
        lossp_ref[0, g] = jnp.concatenate(losses[g], axis=1)


def kernel(x, W_in, b_in, W_out, b_out, codebooks):
    xr = x.reshape(N, DIM)
    winT = W_in.transpose(0, 2, 1)
    woutT = W_out.transpose(0, 2, 1)
    cbT = codebooks.transpose(0, 1, 3, 2)
    bin3 = b_in.reshape(GROUPS, 1, CB_DIM)
    bout3 = b_out.reshape(GROUPS, 1, GDIM)

    q, c, lossp = pl.pallas_call(
        _grvq_kernel,
        grid=(NT,),
        in_specs=[
            pl.BlockSpec((T, DIM), lambda t: (t, 0)),
            pl.BlockSpec((GROUPS, GDIM, CB_DIM), lambda t: (0, 0, 0)),
            pl.BlockSpec((GROUPS, 1, CB_DIM), lambda t: (0, 0, 0)),
            pl.BlockSpec((GROUPS, CB_DIM, GDIM), lambda t: (0, 0, 0)),
            pl.BlockSpec((GROUPS, 1, GDIM), lambda t: (0, 0, 0)),
            pl.BlockSpec((GROUPS, NQ, CB_SIZE, CB_DIM), lambda t: (0, 0, 0, 0)),
            pl.BlockSpec((GROUPS, NQ, CB_DIM, CB_SIZE), lambda t: (0, 0, 0, 0)),
        ],
        out_specs=[
            pl.BlockSpec((T, DIM), lambda t: (t, 0)),
            pl.BlockSpec((GROUPS, T, NQ), lambda t: (0, t, 0)),
            pl.BlockSpec((1, GROUPS, 1, NQ), lambda t: (t, 0, 0, 0)),
        ],
        out_shape=[
            jax.ShapeDtypeStruct((N, DIM), jnp.float32),
            jax.ShapeDtypeStruct((GROUPS, N, NQ), jnp.int32),
            jax.ShapeDtypeStruct((NT, GROUPS, 1, NQ), jnp.float32),
        ],
        scratch_shapes=[pltpu.VMEM((GROUPS, NQ, 1, CB_SIZE), jnp.float32)],
    )(xr, winT, bin3, woutT, bout3, codebooks, cbT)

    qg = q.reshape(B, L, DIM)
    cf = c.reshape(GROUPS, B, L, NQ)
    commit = lossp.sum(axis=0).reshape(GROUPS, NQ) / jnp.float32(N * CB_DIM)
    return (qg, cf, commit)
